# Initial kernel scaffold; baseline (speedup 1.0000x reference)
import jax
import jax.numpy as jnp
from jax.experimental import pallas as pl


def kernel(x, edge_index, edge_attr, enc_W, enc_b, sage_Wl, sage_bl, sage_Wr, bn_g, bn_b, ee_W1, ee_b1, ee_W2, ee_b2, ep_W1, ep_b1, ep_W2, ep_b2, ep_W3, ep_b3):
    raise NotImplementedError("write your pallas kernel here")


# speedup vs baseline: 3.8665x; 3.8665x over previous
import functools

import jax
import jax.numpy as jnp
from jax import lax
from jax.experimental import pallas as pl
from jax.experimental.pallas import tpu as pltpu
from jax.experimental.pallas import tpu_sc as plsc

N = 10000
E = 320000
D = 128
DE = 16
H = 128

NC = 2
NS = 16
NW = NC * NS
EPW = E // NW
CH = 80
NCHUNK = EPW // CH
BCH = 25
NBLK = NCHUNK // BCH
RPT = 640
NP = NS * RPT
ZB = 32

TE = 2560

_f32 = jnp.float32

_mesh = plsc.VectorSubcoreMesh(
    core_axis_name="c", subcore_axis_name="s", num_cores=NC, num_subcores=NS)


def _make_agg(w):
  out_type = [jax.ShapeDtypeStruct((NC * NP, w), _f32)]
  scratch = [
      pltpu.VMEM_SHARED((NP, w), _f32),
      pltpu.VMEM((BCH, CH), jnp.int32),
      pltpu.VMEM((BCH, CH), jnp.int32),
      pltpu.VMEM((CH, w), _f32),
      pltpu.SemaphoreType.DMA,
  ]

  def body(h_hbm, src_hbm, dst_hbm, zr_hbm, out_hbm, acc, sidx, didx,
           rows, sem):
    c = lax.axis_index("c")
    s = lax.axis_index("s")
    wid = c * NS + s
    roff = s * RPT

    def zcp(r, _):
      pltpu.sync_copy(zr_hbm, acc.at[pl.ds(roff + r * ZB, ZB)])
      return 0
    lax.fori_loop(0, RPT // ZB, zcp, 0)

    plsc.subcore_barrier()

    def blk(b, _):
      pltpu.sync_copy(src_hbm.at[wid, b], sidx)
      pltpu.sync_copy(dst_hbm.at[wid, b], didx)

      def step(j, _):
        pltpu.async_copy(h_hbm.at[sidx.at[j]], rows, sem).wait()
        pltpu.sync_copy(rows, acc.at[didx.at[j]], add=True)
        return 0
      return lax.fori_loop(0, BCH, step, 0)
    lax.fori_loop(0, NBLK, blk, 0)

    plsc.subcore_barrier()

    pltpu.sync_copy(acc.at[pl.ds(roff, RPT)],
                    out_hbm.at[pl.ds(c * NP + roff, RPT)])

  return functools.partial(
      pl.kernel, mesh=_mesh, out_type=out_type, scratch_types=scratch)(body)


_agg = _make_agg(H)


@functools.partial(
    pl.kernel, mesh=_mesh,
    out_type=[jax.ShapeDtypeStruct((NC * NP, H), _f32)],
    scratch_types=[
        pltpu.VMEM_SHARED((NP, H), _f32),
        pltpu.VMEM((BCH, CH), jnp.int32),
        pltpu.VMEM((CH, H), _f32),
    ])
def _deg(dst_hbm, zr_hbm, one_hbm, out_hbm, dacc, didx, ones):
  c = lax.axis_index("c")
  s = lax.axis_index("s")
  wid = c * NS + s
  roff = s * RPT

  def zcp(r, _):
    pltpu.sync_copy(zr_hbm, dacc.at[pl.ds(roff + r * ZB, ZB)])
    return 0
  lax.fori_loop(0, RPT // ZB, zcp, 0)
  pltpu.sync_copy(one_hbm, ones)

  plsc.subcore_barrier()

  def blk(b, _):
    pltpu.sync_copy(dst_hbm.at[wid, b], didx)

    def step(j, _):
      pltpu.sync_copy(ones, dacc.at[didx.at[j]], add=True)
      return 0
    return lax.fori_loop(0, BCH, step, 0)
  lax.fori_loop(0, NBLK, blk, 0)

  plsc.subcore_barrier()

  pltpu.sync_copy(dacc.at[pl.ds(roff, RPT)],
                  out_hbm.at[pl.ds(c * NP + roff, RPT)])


@functools.partial(
    pl.kernel, mesh=_mesh,
    out_type=[jax.ShapeDtypeStruct((E, H), _f32),
              jax.ShapeDtypeStruct((E, H), _f32)],
    scratch_types=[
        pltpu.VMEM((CH,), jnp.int32),
        pltpu.VMEM((CH,), jnp.int32),
        pltpu.VMEM((CH, H), _f32),
        pltpu.VMEM((CH, H), _f32),
        pltpu.SemaphoreType.DMA,
        pltpu.SemaphoreType.DMA,
    ])
def _pairgather(ha_hbm, hb_hbm, src_hbm, dst_hbm, ga_hbm, gb_hbm,
                sidx, didx, bufa, bufb, sema, semb):
  c = lax.axis_index("c")
  s = lax.axis_index("s")
  wid = c * NS + s

  def step(j, _):
    base = wid * EPW + j * CH
    pltpu.sync_copy(src_hbm.at[pl.ds(base, CH)], sidx)
    pltpu.sync_copy(dst_hbm.at[pl.ds(base, CH)], didx)
    cpa = pltpu.async_copy(ha_hbm.at[sidx], bufa, sema)
    cpb = pltpu.async_copy(hb_hbm.at[didx], bufb, semb)
    cpa.wait()
    cpb.wait()
    pltpu.sync_copy(bufa, ga_hbm.at[pl.ds(base, CH)])
    pltpu.sync_copy(bufb, gb_hbm.at[pl.ds(base, CH)])
    return 0
  lax.fori_loop(0, NCHUNK, step, 0)


def _prep_body(x_ref, ew_ref, eb_ref, eew2_ref, w1c_ref, eeb2_ref, epb1_ref,
               h_ref, m_ref, cv_ref):
  h_ref[:] = jnp.maximum(
      jnp.dot(x_ref[:], ew_ref[:], preferred_element_type=_f32) + eb_ref[:],
      0.0)
  m_ref[:] = jnp.dot(eew2_ref[:], w1c_ref[:], preferred_element_type=_f32)
  cv_ref[:] = (jnp.dot(eeb2_ref[:], w1c_ref[:], preferred_element_type=_f32)
               + epb1_ref[:])


_prep = pl.pallas_call(
    _prep_body,
    out_shape=[
        jax.ShapeDtypeStruct((N, H), _f32),
        jax.ShapeDtypeStruct((H, H), _f32),
        jax.ShapeDtypeStruct((1, H), _f32),
    ])


def _sage_update(aggp_ref, degp_ref, h_ref, wl_ref, bl_ref, wr_ref,
                 g_ref, b_ref):
  agg = aggp_ref[0] + aggp_ref[1]
  dg = degp_ref[0] + degp_ref[1]
  invd = 1.0 / jnp.maximum(dg[:, 0:1], 1.0)
  hn = (jnp.dot(agg * invd, wl_ref[:], preferred_element_type=_f32)
        + bl_ref[:]
        + jnp.dot(h_ref[:], wr_ref[:], preferred_element_type=_f32))
  mu = jnp.mean(hn, axis=0, keepdims=True)
  var = jnp.mean((hn - mu) ** 2, axis=0, keepdims=True)
  hn = (hn - mu) / jnp.sqrt(var + 1e-5) * g_ref[:] + b_ref[:]
  return h_ref[:] + jnp.maximum(hn, 0.0)


def _layer_body(aggp_ref, degp_ref, h_ref, wl_ref, bl_ref, wr_ref,
                g_ref, b_ref, out_ref):
  out_ref[:] = _sage_update(aggp_ref, degp_ref, h_ref, wl_ref, bl_ref,
                            wr_ref, g_ref, b_ref)


def _layer_final_body(aggp_ref, degp_ref, h_ref, wl_ref, bl_ref, wr_ref,
                      g_ref, b_ref, w1a_ref, w1b_ref,
                      out_ref, ha_ref, hb_ref):
  hnew = _sage_update(aggp_ref, degp_ref, h_ref, wl_ref, bl_ref, wr_ref,
                      g_ref, b_ref)
  out_ref[:] = hnew
  ha_ref[:] = jnp.dot(hnew, w1a_ref[:], preferred_element_type=_f32)
  hb_ref[:] = jnp.dot(hnew, w1b_ref[:], preferred_element_type=_f32)


_layer = pl.pallas_call(
    _layer_body, out_shape=jax.ShapeDtypeStruct((N, H), _f32))

_layer_final = pl.pallas_call(
    _layer_final_body,
    out_shape=[
        jax.ShapeDtypeStruct((N, H), _f32),
        jax.ShapeDtypeStruct((N, H), _f32),
        jax.ShapeDtypeStruct((N, H), _f32),
    ])


def _edge_body(ga_ref, gb_ref, ea_ref, eew1_ref, eeb1_ref, m_ref, cv_ref,
               w2_ref, b2_ref, w3_ref, b3_ref, out_ref):
  t = jnp.maximum(
      jnp.dot(ea_ref[:], eew1_ref[:], preferred_element_type=_f32)
      + eeb1_ref[:], 0.0)
  z = jnp.maximum(
      ga_ref[:] + gb_ref[:]
      + jnp.dot(t, m_ref[:], preferred_element_type=_f32)
      + cv_ref[:], 0.0)
  z2 = jnp.maximum(
      jnp.dot(z, w2_ref[:], preferred_element_type=_f32) + b2_ref[:], 0.0)
  o = jnp.sum(z2 * w3_ref[:], axis=1, keepdims=True) + b3_ref[0, 0]
  out_ref[:] = 1.0 / (1.0 + jnp.exp(-o))


_edge = pl.pallas_call(
    _edge_body,
    grid=(E // TE,),
    in_specs=[
        pl.BlockSpec((TE, H), lambda i: (i, 0)),
        pl.BlockSpec((TE, H), lambda i: (i, 0)),
        pl.BlockSpec((TE, DE), lambda i: (i, 0)),
        pl.BlockSpec((DE, H), lambda i: (0, 0)),
        pl.BlockSpec((1, H), lambda i: (0, 0)),
        pl.BlockSpec((H, H), lambda i: (0, 0)),
        pl.BlockSpec((1, H), lambda i: (0, 0)),
        pl.BlockSpec((H, H // 2), lambda i: (0, 0)),
        pl.BlockSpec((1, H // 2), lambda i: (0, 0)),
        pl.BlockSpec((1, H // 2), lambda i: (0, 0)),
        pl.BlockSpec((1, 1), lambda i: (0, 0)),
    ],
    out_specs=pl.BlockSpec((TE, 1), lambda i: (i, 0)),
    out_shape=jax.ShapeDtypeStruct((E, 1), _f32),
)


def kernel(x, edge_index, edge_attr, enc_W, enc_b, sage_Wl, sage_bl,
           sage_Wr, bn_g, bn_b, ee_W1, ee_b1, ee_W2, ee_b2, ep_W1, ep_b1,
           ep_W2, ep_b2, ep_W3, ep_b3):
  src = edge_index[0]
  dst = edge_index[1]
  src3 = src.reshape(NW, NBLK, BCH, CH)
  dst3 = dst.reshape(NW, NBLK, BCH, CH)
  w1a = ep_W1[:H]
  w1b = ep_W1[H:2 * H]
  w1c = ep_W1[2 * H:]

  h, m, cv = _prep(x, enc_W, enc_b.reshape(1, H), ee_W2, w1c,
                   ee_b2.reshape(1, H), ep_b1.reshape(1, H))

  zrow = jnp.zeros((ZB, H), _f32)
  onerow = jnp.ones((CH, H), _f32)

  (degp,) = _deg(dst3, zrow, onerow)
  degp = degp.reshape(NC, NP, H)[:, :N, :16]
  (aggp,) = _agg(h, src3, dst3, zrow)
  aggp = aggp.reshape(NC, NP, H)[:, :N]
  h = _layer(aggp, degp, h, sage_Wl[0], sage_bl[0].reshape(1, H),
             sage_Wr[0], bn_g[0].reshape(1, H), bn_b[0].reshape(1, H))

  (aggp,) = _agg(h, src3, dst3, zrow)
  aggp = aggp.reshape(NC, NP, H)[:, :N]
  h = _layer(aggp, degp, h, sage_Wl[1], sage_bl[1].reshape(1, H),
             sage_Wr[1], bn_g[1].reshape(1, H), bn_b[1].reshape(1, H))

  (aggp,) = _agg(h, src3, dst3, zrow)
  aggp = aggp.reshape(NC, NP, H)[:, :N]
  h, ha, hb = _layer_final(
      aggp, degp, h, sage_Wl[2], sage_bl[2].reshape(1, H), sage_Wr[2],
      bn_g[2].reshape(1, H), bn_b[2].reshape(1, H), w1a, w1b)

  ga, gb = _pairgather(ha, hb, src, dst)

  out = _edge(ga, gb, edge_attr, ee_W1, ee_b1.reshape(1, H), m, cv, ep_W2,
              ep_b2.reshape(1, H // 2), ep_W3.reshape(1, H // 2),
              ep_b3.reshape(1, 1))
  return out.reshape(E)


# speedup vs baseline: 5.0790x; 1.3136x over previous
import functools

import jax
import jax.numpy as jnp
from jax import lax
from jax.experimental import pallas as pl
from jax.experimental.pallas import tpu as pltpu
from jax.experimental.pallas import tpu_sc as plsc

N = 10000
E = 320000
D = 128
DE = 16
H = 128

NC = 2
NS = 16
NW = NC * NS
EPW = E // NW
CH = 80
NCHUNK = EPW // CH
NPAIR = NCHUNK // 2
BPAIR = 12
BCH = 25
NBLK = NCHUNK // BCH
RPT = 640
NP = NS * RPT
ZB = 32

TE = 2560

_f32 = jnp.float32

_mesh = plsc.VectorSubcoreMesh(
    core_axis_name="c", subcore_axis_name="s", num_cores=NC, num_subcores=NS)


def _make_agg(w):
  out_type = [jax.ShapeDtypeStruct((NC * NP, w), _f32)]
  scratch = [
      pltpu.VMEM_SHARED((NP, w), _f32),
      pltpu.VMEM((BCH, CH), jnp.int32),
      pltpu.VMEM((BCH, CH), jnp.int32),
      pltpu.VMEM((CH, w), _f32),
      pltpu.VMEM((CH, w), _f32),
      pltpu.SemaphoreType.DMA,
      pltpu.SemaphoreType.DMA,
  ]

  def body(h_hbm, src_hbm, dst_hbm, zr_hbm, out_hbm, acc, sidx, didx,
           rows0, rows1, sem0, sem1):
    c = lax.axis_index("c")
    s = lax.axis_index("s")
    wid = c * NS + s
    roff = s * RPT

    def zcp(r, _):
      pltpu.sync_copy(zr_hbm, acc.at[pl.ds(roff + r * ZB, ZB)])
      return 0
    lax.fori_loop(0, RPT // ZB, zcp, 0)

    plsc.subcore_barrier()

    def blk(b, _):
      pltpu.sync_copy(src_hbm.at[wid, b], sidx)
      pltpu.sync_copy(dst_hbm.at[wid, b], didx)

      pltpu.async_copy(h_hbm.at[sidx.at[0]], rows0, sem0)
      pltpu.async_copy(h_hbm.at[sidx.at[1]], rows1, sem1)

      def pair(i, _):
        j0 = 2 * i
        pltpu.make_async_copy(h_hbm.at[sidx.at[j0]], rows0, sem0).wait()
        pltpu.sync_copy(rows0, acc.at[didx.at[j0]], add=True)
        pltpu.async_copy(h_hbm.at[sidx.at[j0 + 2]], rows0, sem0)

        @pl.when(i < BPAIR - 1)
        def _():
          pltpu.make_async_copy(h_hbm.at[sidx.at[j0 + 1]], rows1,
                                sem1).wait()
          pltpu.sync_copy(rows1, acc.at[didx.at[j0 + 1]], add=True)
          pltpu.async_copy(h_hbm.at[sidx.at[j0 + 3]], rows1, sem1)
        return 0
      lax.fori_loop(0, BPAIR, pair, 0)

      pltpu.make_async_copy(h_hbm.at[sidx.at[BCH - 2]], rows1, sem1).wait()
      pltpu.sync_copy(rows1, acc.at[didx.at[BCH - 2]], add=True)
      pltpu.make_async_copy(h_hbm.at[sidx.at[BCH - 1]], rows0, sem0).wait()
      pltpu.sync_copy(rows0, acc.at[didx.at[BCH - 1]], add=True)
      return 0
    lax.fori_loop(0, NBLK, blk, 0)

    plsc.subcore_barrier()

    pltpu.sync_copy(acc.at[pl.ds(roff, RPT)],
                    out_hbm.at[pl.ds(c * NP + roff, RPT)])

  return functools.partial(
      pl.kernel, mesh=_mesh, out_type=out_type, scratch_types=scratch)(body)


_agg = _make_agg(H)


@functools.partial(
    pl.kernel, mesh=_mesh,
    out_type=[jax.ShapeDtypeStruct((NC * NP, H), _f32)],
    scratch_types=[
        pltpu.VMEM_SHARED((NP, H), _f32),
        pltpu.VMEM((BCH, CH), jnp.int32),
        pltpu.VMEM((CH, H), _f32),
    ])
def _deg(dst_hbm, zr_hbm, one_hbm, out_hbm, dacc, didx, ones):
  c = lax.axis_index("c")
  s = lax.axis_index("s")
  wid = c * NS + s
  roff = s * RPT

  def zcp(r, _):
    pltpu.sync_copy(zr_hbm, dacc.at[pl.ds(roff + r * ZB, ZB)])
    return 0
  lax.fori_loop(0, RPT // ZB, zcp, 0)
  pltpu.sync_copy(one_hbm, ones)

  plsc.subcore_barrier()

  def blk(b, _):
    pltpu.sync_copy(dst_hbm.at[wid, b], didx)

    def step(j, _):
      pltpu.sync_copy(ones, dacc.at[didx.at[j]], add=True)
      return 0
    return lax.fori_loop(0, BCH, step, 0)
  lax.fori_loop(0, NBLK, blk, 0)

  plsc.subcore_barrier()

  pltpu.sync_copy(dacc.at[pl.ds(roff, RPT)],
                  out_hbm.at[pl.ds(c * NP + roff, RPT)])


@functools.partial(
    pl.kernel, mesh=_mesh,
    out_type=[jax.ShapeDtypeStruct((E, H), _f32),
              jax.ShapeDtypeStruct((E, H), _f32)],
    scratch_types=[
        pltpu.VMEM((NCHUNK, CH), jnp.int32),
        pltpu.VMEM((NCHUNK, CH), jnp.int32),
        pltpu.VMEM((CH, H), _f32),
        pltpu.VMEM((CH, H), _f32),
        pltpu.VMEM((CH, H), _f32),
        pltpu.VMEM((CH, H), _f32),
        pltpu.SemaphoreType.DMA,
        pltpu.SemaphoreType.DMA,
        pltpu.SemaphoreType.DMA,
        pltpu.SemaphoreType.DMA,
    ])
def _pairgather(ha_hbm, hb_hbm, src_hbm, dst_hbm, ga_hbm, gb_hbm,
                sidx, didx, bufa0, bufb0, bufa1, bufb1,
                sema0, semb0, sema1, semb1):
  c = lax.axis_index("c")
  s = lax.axis_index("s")
  wid = c * NS + s
  wbase = wid * EPW

  pltpu.sync_copy(src_hbm.at[wid], sidx)
  pltpu.sync_copy(dst_hbm.at[wid], didx)

  def fire(j, ba, bb, sa, sb):
    pltpu.async_copy(ha_hbm.at[sidx.at[j]], ba, sa)
    pltpu.async_copy(hb_hbm.at[didx.at[j]], bb, sb)

  def drain_write(j, ba, bb, sa, sb):
    pltpu.make_async_copy(ha_hbm.at[sidx.at[j]], ba, sa).wait()
    pltpu.make_async_copy(hb_hbm.at[didx.at[j]], bb, sb).wait()
    pltpu.sync_copy(ba, ga_hbm.at[pl.ds(wbase + j * CH, CH)])
    pltpu.sync_copy(bb, gb_hbm.at[pl.ds(wbase + j * CH, CH)])

  fire(0, bufa0, bufb0, sema0, semb0)
  fire(1, bufa1, bufb1, sema1, semb1)

  def pair(i, _):
    j0 = 2 * i
    drain_write(j0, bufa0, bufb0, sema0, semb0)
    fire(j0 + 2, bufa0, bufb0, sema0, semb0)

    @pl.when(i < NPAIR - 1)
    def _():
      drain_write(j0 + 1, bufa1, bufb1, sema1, semb1)
      fire(j0 + 3, bufa1, bufb1, sema1, semb1)
    return 0
  lax.fori_loop(0, NPAIR, pair, 0)

  drain_write(NCHUNK - 2, bufa1, bufb1, sema1, semb1)
  drain_write(NCHUNK - 1, bufa0, bufb0, sema0, semb0)


def _prep_body(x_ref, ew_ref, eb_ref, eew2_ref, w1c_ref, eeb2_ref, epb1_ref,
               h_ref, m_ref, cv_ref):
  h_ref[:] = jnp.maximum(
      jnp.dot(x_ref[:], ew_ref[:], preferred_element_type=_f32) + eb_ref[:],
      0.0)
  m_ref[:] = jnp.dot(eew2_ref[:], w1c_ref[:], preferred_element_type=_f32)
  cv_ref[:] = (jnp.dot(eeb2_ref[:], w1c_ref[:], preferred_element_type=_f32)
               + epb1_ref[:])


_prep = pl.pallas_call(
    _prep_body,
    out_shape=[
        jax.ShapeDtypeStruct((N, H), _f32),
        jax.ShapeDtypeStruct((H, H), _f32),
        jax.ShapeDtypeStruct((1, H), _f32),
    ])


def _sage_update(aggp_ref, degp_ref, h_ref, wl_ref, bl_ref, wr_ref,
                 g_ref, b_ref):
  agg = aggp_ref[0] + aggp_ref[1]
  dg = degp_ref[0] + degp_ref[1]
  invd = 1.0 / jnp.maximum(dg[:, 0:1], 1.0)
  hn = (jnp.dot(agg * invd, wl_ref[:], preferred_element_type=_f32)
        + bl_ref[:]
        + jnp.dot(h_ref[:], wr_ref[:], preferred_element_type=_f32))
  mu = jnp.mean(hn, axis=0, keepdims=True)
  var = jnp.mean((hn - mu) ** 2, axis=0, keepdims=True)
  hn = (hn - mu) / jnp.sqrt(var + 1e-5) * g_ref[:] + b_ref[:]
  return h_ref[:] + jnp.maximum(hn, 0.0)


def _layer_body(aggp_ref, degp_ref, h_ref, wl_ref, bl_ref, wr_ref,
                g_ref, b_ref, out_ref):
  out_ref[:] = _sage_update(aggp_ref, degp_ref, h_ref, wl_ref, bl_ref,
                            wr_ref, g_ref, b_ref)


def _layer_final_body(aggp_ref, degp_ref, h_ref, wl_ref, bl_ref, wr_ref,
                      g_ref, b_ref, w1a_ref, w1b_ref,
                      out_ref, ha_ref, hb_ref):
  hnew = _sage_update(aggp_ref, degp_ref, h_ref, wl_ref, bl_ref, wr_ref,
                      g_ref, b_ref)
  out_ref[:] = hnew
  ha_ref[:] = jnp.dot(hnew, w1a_ref[:], preferred_element_type=_f32)
  hb_ref[:] = jnp.dot(hnew, w1b_ref[:], preferred_element_type=_f32)


_layer = pl.pallas_call(
    _layer_body, out_shape=jax.ShapeDtypeStruct((N, H), _f32))

_layer_final = pl.pallas_call(
    _layer_final_body,
    out_shape=[
        jax.ShapeDtypeStruct((N, H), _f32),
        jax.ShapeDtypeStruct((N, H), _f32),
        jax.ShapeDtypeStruct((N, H), _f32),
    ])


def _edge_body(ga_ref, gb_ref, ea_ref, eew1_ref, eeb1_ref, m_ref, cv_ref,
               w2_ref, b2_ref, w3_ref, b3_ref, out_ref):
  t = jnp.maximum(
      jnp.dot(ea_ref[:], eew1_ref[:], preferred_element_type=_f32)
      + eeb1_ref[:], 0.0)
  z = jnp.maximum(
      ga_ref[:] + gb_ref[:]
      + jnp.dot(t, m_ref[:], preferred_element_type=_f32)
      + cv_ref[:], 0.0)
  z2 = jnp.maximum(
      jnp.dot(z, w2_ref[:], preferred_element_type=_f32) + b2_ref[:], 0.0)
  o = jnp.sum(z2 * w3_ref[:], axis=1, keepdims=True) + b3_ref[0, 0]
  out_ref[:] = 1.0 / (1.0 + jnp.exp(-o))


_edge = pl.pallas_call(
    _edge_body,
    grid=(E // TE,),
    in_specs=[
        pl.BlockSpec((TE, H), lambda i: (i, 0)),
        pl.BlockSpec((TE, H), lambda i: (i, 0)),
        pl.BlockSpec((TE, DE), lambda i: (i, 0)),
        pl.BlockSpec((DE, H), lambda i: (0, 0)),
        pl.BlockSpec((1, H), lambda i: (0, 0)),
        pl.BlockSpec((H, H), lambda i: (0, 0)),
        pl.BlockSpec((1, H), lambda i: (0, 0)),
        pl.BlockSpec((H, H // 2), lambda i: (0, 0)),
        pl.BlockSpec((1, H // 2), lambda i: (0, 0)),
        pl.BlockSpec((1, H // 2), lambda i: (0, 0)),
        pl.BlockSpec((1, 1), lambda i: (0, 0)),
    ],
    out_specs=pl.BlockSpec((TE, 1), lambda i: (i, 0)),
    out_shape=jax.ShapeDtypeStruct((E, 1), _f32),
)


def kernel(x, edge_index, edge_attr, enc_W, enc_b, sage_Wl, sage_bl,
           sage_Wr, bn_g, bn_b, ee_W1, ee_b1, ee_W2, ee_b2, ep_W1, ep_b1,
           ep_W2, ep_b2, ep_W3, ep_b3):
  src = edge_index[0]
  dst = edge_index[1]
  src4 = src.reshape(NW, NCHUNK, CH)
  dst4 = dst.reshape(NW, NCHUNK, CH)
  src3 = src.reshape(NW, NBLK, BCH, CH)
  dst3 = dst.reshape(NW, NBLK, BCH, CH)
  w1a = ep_W1[:H]
  w1b = ep_W1[H:2 * H]
  w1c = ep_W1[2 * H:]

  h, m, cv = _prep(x, enc_W, enc_b.reshape(1, H), ee_W2, w1c,
                   ee_b2.reshape(1, H), ep_b1.reshape(1, H))

  zrow = jnp.zeros((ZB, H), _f32)
  onerow = jnp.ones((CH, H), _f32)

  (degp,) = _deg(dst3, zrow, onerow)
  degp = degp.reshape(NC, NP, H)[:, :N, :16]
  (aggp,) = _agg(h, src3, dst3, zrow)
  aggp = aggp.reshape(NC, NP, H)[:, :N]
  h = _layer(aggp, degp, h, sage_Wl[0], sage_bl[0].reshape(1, H),
             sage_Wr[0], bn_g[0].reshape(1, H), bn_b[0].reshape(1, H))

  (aggp,) = _agg(h, src3, dst3, zrow)
  aggp = aggp.reshape(NC, NP, H)[:, :N]
  h = _layer(aggp, degp, h, sage_Wl[1], sage_bl[1].reshape(1, H),
             sage_Wr[1], bn_g[1].reshape(1, H), bn_b[1].reshape(1, H))

  (aggp,) = _agg(h, src3, dst3, zrow)
  aggp = aggp.reshape(NC, NP, H)[:, :N]
  h, ha, hb = _layer_final(
      aggp, degp, h, sage_Wl[2], sage_bl[2].reshape(1, H), sage_Wr[2],
      bn_g[2].reshape(1, H), bn_b[2].reshape(1, H), w1a, w1b)

  ga, gb = _pairgather(ha, hb, src4, dst4)

  out = _edge(ga, gb, edge_attr, ee_W1, ee_b1.reshape(1, H), m, cv, ep_W2,
              ep_b2.reshape(1, H // 2), ep_W3.reshape(1, H // 2),
              ep_b3.reshape(1, 1))
  return out.reshape(E)
